# Initial kernel scaffold; baseline (speedup 1.0000x reference)
import jax
import jax.numpy as jnp
from jax.experimental import pallas as pl


def kernel(coord, feat, offset, reference_index_self_attn, skip_coord, skip_feat, skip_offset, reference_index_cross_attn, params):
    raise NotImplementedError("write your pallas kernel here")


# speedup vs baseline: 4.1396x; 4.1396x over previous
import functools

import numpy as np
import jax
import jax.numpy as jnp
from jax import lax
from jax.experimental import pallas as pl
from jax.experimental.pallas import tpu as pltpu
from jax.experimental.pallas import tpu_sc as plsc

_C = 128
_G = 8
_NS = 32
_BQ = 80
_BP = 2000
_CH = 80


def _mm(x, w):
    return lax.dot_general(x, w, (((x.ndim - 1,), (0,)), ((), ())),
                           preferred_element_type=jnp.float32)


def _ln(x, g, b, eps=1e-5):
    m = jnp.mean(x, axis=-1, keepdims=True)
    v = jnp.mean((x - m) ** 2, axis=-1, keepdims=True)
    return (x - m) / jnp.sqrt(v + eps) * g + b


def _relu(x):
    return jnp.maximum(x, 0.0)


def _proj_body(feat, skip, xyz16, sxyz16,
               qW, qb, qg, qlb, kW, kb, kg, klb, vW, vb, w1,
               ckW, ckb, ckg, cklb, cvW, cvb, cw1,
               q8_o, auxs_o, vs_o, auxc_o, vc_o):
    f = feat[...]
    s = skip[...]
    q = _relu(_ln(_mm(f, qW[...]) + qb[...], qg[...], qlb[...]))
    k = _relu(_ln(_mm(f, kW[...]) + kb[...], kg[...], klb[...]))
    q8_o[...] = _mm(q, w1[...])
    z8 = jnp.zeros((f.shape[0], _G), jnp.float32)
    auxs_o[...] = jnp.concatenate([xyz16[...], _mm(k, w1[...]), z8], axis=1)
    vs_o[...] = _mm(f, vW[...]) + vb[...]
    ck = _relu(_ln(_mm(s, ckW[...]) + ckb[...], ckg[...], cklb[...]))
    auxc_o[...] = jnp.concatenate([sxyz16[...], _mm(ck, cw1[...]), z8], axis=1)
    vc_o[...] = _mm(s, cvW[...]) + cvb[...]


def _proj(feat, skip, xyz16, sxyz16, sa, ca):
    n = feat.shape[0]
    grid = n // _BP
    row = lambda d: pl.BlockSpec((_BP, d), lambda i: (i, 0))
    full = lambda a: pl.BlockSpec(a.shape, lambda i: (0,) * a.ndim)
    params = [sa["q"]["W"].T, sa["q"]["b"][None], sa["q_ln"]["g"][None], sa["q_ln"]["b"][None],
              sa["k"]["W"].T, sa["k"]["b"][None], sa["k_ln"]["g"][None], sa["k_ln"]["b"][None],
              sa["v"]["W"].T, sa["v"]["b"][None], sa["we1"]["W"].T,
              ca["k"]["W"].T, ca["k"]["b"][None], ca["k_ln"]["g"][None], ca["k_ln"]["b"][None],
              ca["v"]["W"].T, ca["v"]["b"][None], ca["we1"]["W"].T]
    return pl.pallas_call(
        _proj_body,
        grid=(grid,),
        in_specs=[row(_C), row(_C), row(16), row(16)] + [full(p) for p in params],
        out_specs=[row(_G), row(32), row(_C), row(32), row(_C)],
        out_shape=[jax.ShapeDtypeStruct((n, _G), jnp.float32),
                   jax.ShapeDtypeStruct((n, 32), jnp.float32),
                   jax.ShapeDtypeStruct((n, _C), jnp.float32),
                   jax.ShapeDtypeStruct((n, 32), jnp.float32),
                   jax.ShapeDtypeStruct((n, _C), jnp.float32)],
    )(feat, skip, xyz16, sxyz16, *params)


def _sc_gather(aux, vtab, idx_flat):
    r = idx_flat.shape[0]
    info = plsc.get_sparse_core_info()
    nw = info.num_cores * info.num_subcores
    rw = r // nw
    iters = rw // _CH
    mesh = plsc.VectorSubcoreMesh(core_axis_name="c", subcore_axis_name="s")

    @functools.partial(
        pl.kernel, mesh=mesh,
        out_type=[jax.ShapeDtypeStruct((r, 32), jnp.float32),
                  jax.ShapeDtypeStruct((r, _C), jnp.float32)],
        scratch_types=[pltpu.VMEM((_CH,), jnp.int32),
                       pltpu.VMEM((_CH, 32), jnp.float32),
                       pltpu.VMEM((_CH, _C), jnp.float32),
                       pltpu.SemaphoreType.DMA],
        compiler_params=pltpu.CompilerParams(use_tc_tiling_on_sc=False),
    )
    def gk(aux_h, vtab_h, idx_h, gaux_h, gval_h, idx_v, abuf, vbuf, sem):
        wid = lax.axis_index("s") * info.num_cores + lax.axis_index("c")
        base = wid * rw

        def step(i, carry):
            off = base + i * _CH
            pltpu.sync_copy(idx_h.at[pl.ds(off, _CH)], idx_v)
            ca = pltpu.async_copy(aux_h.at[idx_v], abuf, sem)
            cv = pltpu.async_copy(vtab_h.at[idx_v], vbuf, sem)
            ca.wait()
            cv.wait()
            pltpu.sync_copy(abuf, gaux_h.at[pl.ds(off, _CH)])
            pltpu.sync_copy(vbuf, gval_h.at[pl.ds(off, _CH)])
            return carry

        lax.fori_loop(0, iters, step, 0)

    return gk(aux, vtab, idx_flat)


def _attn_body(q8, gaux, gval, auxq,
               p1W, p1b, pg, pb_, p2W, p2b, w1, w1b, wg, wb, w2, w2b,
               sel, ex, out):
    bq = q8.shape[0]
    ga = gaux[...]
    a = _mm(ga, p1W[...]) + p1b[...]
    bqr = _mm(auxq[...], p1W[...])
    t3 = a.reshape(bq, _NS, _C) - bqr[:, None, :]
    t3 = _relu(_ln(t3, pg[...], pb_[...]))
    t = t3.reshape(bq * _NS, _C)
    peb = _mm(t, p2W[...]) + p2b[...]
    u8 = _mm(peb, w1[...]) + _mm(ga, sel[...])
    u3 = u8.reshape(bq, _NS, _G) - q8[...][:, None, :] + w1b[...]
    u3 = _relu(_ln(u3, wg[...], wb[...]))
    w = _mm(u3.reshape(bq * _NS, _G), w2[...]) + w2b[...]
    w3 = w.reshape(bq, _NS, _G)
    w3 = w3 - jnp.max(w3, axis=1, keepdims=True)
    w3 = jnp.exp(w3)
    w3 = w3 / jnp.sum(w3, axis=1, keepdims=True)
    wfull = _mm(w3.reshape(bq * _NS, _G), ex[...])
    acc = (gval[...] + peb) * wfull
    out[...] = jnp.sum(acc.reshape(bq, _NS, _C), axis=1)


def _attn(q8, gaux, gval, auxq, p):
    n = q8.shape[0]
    grid = n // _BQ
    sel = np.zeros((32, _G), np.float32)
    sel[16:16 + _G] = np.eye(_G, dtype=np.float32)
    ex = np.kron(np.eye(_G, dtype=np.float32), np.ones((1, _C // _G), np.float32))
    params = [jnp.concatenate([p["pb1"]["W"].T, jnp.zeros((29, _C), jnp.float32)], axis=0),
              p["pb1"]["b"][None], p["pb_ln"]["g"][None], p["pb_ln"]["b"][None],
              p["pb2"]["W"].T, p["pb2"]["b"][None],
              p["we1"]["W"].T, p["we1"]["b"][None],
              p["we_ln"]["g"][None], p["we_ln"]["b"][None],
              p["we2"]["W"].T, p["we2"]["b"][None], sel, ex]
    full = lambda a: pl.BlockSpec(a.shape, lambda i: (0,) * a.ndim)
    return pl.pallas_call(
        _attn_body,
        grid=(grid,),
        in_specs=[pl.BlockSpec((_BQ, _G), lambda i: (i, 0)),
                  pl.BlockSpec((_BQ * _NS, 32), lambda i: (i, 0)),
                  pl.BlockSpec((_BQ * _NS, _C), lambda i: (i, 0)),
                  pl.BlockSpec((_BQ, 32), lambda i: (i, 0))] + [full(q) for q in params],
        out_specs=pl.BlockSpec((_BQ, _C), lambda i: (i, 0)),
        out_shape=jax.ShapeDtypeStruct((n, _C), jnp.float32),
    )(q8, gaux, gval, auxq, *params)


def _post_sa_body(feat, fsa, n1g, n1b, qW, qb, qg, qlb, w1, feat1_o, q8_o):
    f1 = feat[...] + _ln(fsa[...], n1g[...], n1b[...])
    feat1_o[...] = f1
    q = _relu(_ln(_mm(f1, qW[...]) + qb[...], qg[...], qlb[...]))
    q8_o[...] = _mm(q, w1[...])


def _post_sa(feat, fsa, norm1, ca):
    n = feat.shape[0]
    grid = n // _BP
    row = lambda d: pl.BlockSpec((_BP, d), lambda i: (i, 0))
    full = lambda a: pl.BlockSpec(a.shape, lambda i: (0,) * a.ndim)
    params = [norm1["g"][None], norm1["b"][None],
              ca["q"]["W"].T, ca["q"]["b"][None],
              ca["q_ln"]["g"][None], ca["q_ln"]["b"][None], ca["we1"]["W"].T]
    return pl.pallas_call(
        _post_sa_body,
        grid=(grid,),
        in_specs=[row(_C), row(_C)] + [full(p) for p in params],
        out_specs=[row(_C), row(_G)],
        out_shape=[jax.ShapeDtypeStruct((n, _C), jnp.float32),
                   jax.ShapeDtypeStruct((n, _G), jnp.float32)],
    )(feat, fsa, *params)


def _final_body(feat1, fca, n2g, n2b, m1W, m1b, m2W, m2b, n3g, n3b, out):
    f2 = feat1[...] + _ln(fca[...], n2g[...], n2b[...])
    h = _relu(_mm(f2, m1W[...]) + m1b[...])
    f3 = _ln(_mm(h, m2W[...]) + m2b[...], n3g[...], n3b[...])
    out[...] = _relu(f2 + f3)


def _final(feat1, fca, params):
    n = feat1.shape[0]
    grid = n // _BP
    row = lambda d: pl.BlockSpec((_BP, d), lambda i: (i, 0))
    full = lambda a: pl.BlockSpec(a.shape, lambda i: (0,) * a.ndim)
    ps = [params["norm2"]["g"][None], params["norm2"]["b"][None],
          params["mlp1"]["W"].T, params["mlp1"]["b"][None],
          params["mlp2"]["W"].T, params["mlp2"]["b"][None],
          params["norm3"]["g"][None], params["norm3"]["b"][None]]
    return pl.pallas_call(
        _final_body,
        grid=(grid,),
        in_specs=[row(_C), row(_C)] + [full(p) for p in ps],
        out_specs=row(_C),
        out_shape=jax.ShapeDtypeStruct((n, _C), jnp.float32),
    )(feat1, fca, *ps)


def kernel(coord, feat, offset, reference_index_self_attn, skip_coord,
           skip_feat, skip_offset, reference_index_cross_attn, params):
    n = feat.shape[0]
    z13 = jnp.zeros((n, 13), jnp.float32)
    xyz16 = jnp.concatenate([coord, z13], axis=1)
    sxyz16 = jnp.concatenate([skip_coord, z13], axis=1)

    q8s, auxs, vs, auxc, vc = _proj(feat, skip_feat, xyz16, sxyz16,
                                    params["sa"], params["ca"])

    gaux_s, gval_s = _sc_gather(auxs, vs, reference_index_self_attn.reshape(-1))
    f_sa = _attn(q8s, gaux_s, gval_s, auxs, params["sa"])

    feat1, q8c = _post_sa(feat, f_sa, params["norm1"], params["ca"])

    gaux_c, gval_c = _sc_gather(auxc, vc, reference_index_cross_attn.reshape(-1))
    f_ca = _attn(q8c, gaux_c, gval_c, auxs, params["ca"])

    out_feat = _final(feat1, f_ca, params)
    return (coord, out_feat, offset)


# speedup vs baseline: 5.0110x; 1.2105x over previous
import functools

import numpy as np
import jax
import jax.numpy as jnp
from jax import lax
from jax.experimental import pallas as pl
from jax.experimental.pallas import tpu as pltpu
from jax.experimental.pallas import tpu_sc as plsc

_C = 128
_G = 8
_NS = 32
_BQ = 80
_BP = 2000
_CH = 80


def _mm(x, w):
    return lax.dot_general(x, w, (((x.ndim - 1,), (0,)), ((), ())),
                           preferred_element_type=jnp.float32)


def _ln(x, g, b, eps=1e-5):
    m = jnp.mean(x, axis=-1, keepdims=True)
    v = jnp.mean((x - m) ** 2, axis=-1, keepdims=True)
    return (x - m) / jnp.sqrt(v + eps) * g + b


def _relu(x):
    return jnp.maximum(x, 0.0)


def _proj_body(feat, skip, xyz16, sxyz16,
               qW, qb, qg, qlb, kW, kb, kg, klb, vW, vb, w1,
               ckW, ckb, ckg, cklb, cvW, cvb, cw1,
               q8_o, auxs_o, vs_o, auxc_o, vc_o):
    f = feat[...]
    s = skip[...]
    q = _relu(_ln(_mm(f, qW[...]) + qb[...], qg[...], qlb[...]))
    k = _relu(_ln(_mm(f, kW[...]) + kb[...], kg[...], klb[...]))
    q8_o[...] = _mm(q, w1[...])
    z8 = jnp.zeros((f.shape[0], _G), jnp.float32)
    auxs_o[...] = jnp.concatenate([xyz16[...], _mm(k, w1[...]), z8], axis=1)
    vs_o[...] = _mm(f, vW[...]) + vb[...]
    ck = _relu(_ln(_mm(s, ckW[...]) + ckb[...], ckg[...], cklb[...]))
    auxc_o[...] = jnp.concatenate([sxyz16[...], _mm(ck, cw1[...]), z8], axis=1)
    vc_o[...] = _mm(s, cvW[...]) + cvb[...]


def _proj(feat, skip, xyz16, sxyz16, sa, ca):
    n = feat.shape[0]
    grid = n // _BP
    row = lambda d: pl.BlockSpec((_BP, d), lambda i: (i, 0))
    full = lambda a: pl.BlockSpec(a.shape, lambda i: (0,) * a.ndim)
    params = [sa["q"]["W"].T, sa["q"]["b"][None], sa["q_ln"]["g"][None], sa["q_ln"]["b"][None],
              sa["k"]["W"].T, sa["k"]["b"][None], sa["k_ln"]["g"][None], sa["k_ln"]["b"][None],
              sa["v"]["W"].T, sa["v"]["b"][None], sa["we1"]["W"].T,
              ca["k"]["W"].T, ca["k"]["b"][None], ca["k_ln"]["g"][None], ca["k_ln"]["b"][None],
              ca["v"]["W"].T, ca["v"]["b"][None], ca["we1"]["W"].T]
    return pl.pallas_call(
        _proj_body,
        grid=(grid,),
        in_specs=[row(_C), row(_C), row(16), row(16)] + [full(p) for p in params],
        out_specs=[row(_G), row(32), row(_C), row(32), row(_C)],
        out_shape=[jax.ShapeDtypeStruct((n, _G), jnp.float32),
                   jax.ShapeDtypeStruct((n, 32), jnp.float32),
                   jax.ShapeDtypeStruct((n, _C), jnp.float32),
                   jax.ShapeDtypeStruct((n, 32), jnp.float32),
                   jax.ShapeDtypeStruct((n, _C), jnp.float32)],
    )(feat, skip, xyz16, sxyz16, *params)


_NB = 5


def _sc_gather(aux, vtab, idx_flat):
    r = idx_flat.shape[0]
    info = plsc.get_sparse_core_info()
    nw = info.num_cores * info.num_subcores
    rw = r // nw
    iters = rw // _CH
    outer = iters // _NB
    mesh = plsc.VectorSubcoreMesh(core_axis_name="c", subcore_axis_name="s")

    @functools.partial(
        pl.kernel, mesh=mesh,
        out_type=[jax.ShapeDtypeStruct((r, 32), jnp.float32),
                  jax.ShapeDtypeStruct((r, _C), jnp.float32)],
        scratch_types=[pltpu.VMEM((rw,), jnp.int32),
                       pltpu.VMEM((_NB, _CH, 32), jnp.float32),
                       pltpu.VMEM((_NB, _CH, _C), jnp.float32),
                       pltpu.SemaphoreType.DMA((_NB,)),
                       pltpu.SemaphoreType.DMA((_NB,))],
        compiler_params=pltpu.CompilerParams(use_tc_tiling_on_sc=False),
    )
    def gk(aux_h, vtab_h, idx_h, gaux_h, gval_h, idx_v, abuf, vbuf, semg, semw):
        wid = lax.axis_index("s") * info.num_cores + lax.axis_index("c")
        base = wid * rw
        pltpu.sync_copy(idx_h.at[pl.ds(base, rw)], idx_v)

        def fire(chunk, b):
            sl = idx_v.at[pl.ds(chunk * _CH, _CH)]
            pltpu.async_copy(aux_h.at[sl], abuf.at[b], semg.at[b])
            pltpu.async_copy(vtab_h.at[sl], vbuf.at[b], semg.at[b])

        def wait_gather(chunk, b):
            sl = idx_v.at[pl.ds(chunk * _CH, _CH)]
            pltpu.make_async_copy(aux_h.at[sl], abuf.at[b], semg.at[b]).wait()
            pltpu.make_async_copy(vtab_h.at[sl], vbuf.at[b], semg.at[b]).wait()

        def fire_wb(chunk, b):
            off = base + chunk * _CH
            pltpu.async_copy(abuf.at[b], gaux_h.at[pl.ds(off, _CH)], semw.at[b])
            pltpu.async_copy(vbuf.at[b], gval_h.at[pl.ds(off, _CH)], semw.at[b])

        def wait_wb(chunk, b):
            off = base + chunk * _CH
            pltpu.make_async_copy(abuf.at[b], gaux_h.at[pl.ds(off, _CH)],
                                  semw.at[b]).wait()
            pltpu.make_async_copy(vbuf.at[b], gval_h.at[pl.ds(off, _CH)],
                                  semw.at[b]).wait()

        for b in range(_NB - 1):
            fire(b, b)

        def step(g0, carry):
            for b in range(_NB):
                gb = g0 * _NB + b
                bp = (b - 1) % _NB

                @pl.when(gb > 0)
                def _():
                    wait_wb(gb - 1, bp)

                @pl.when(gb + _NB - 1 < iters)
                def _():
                    fire(gb + _NB - 1, bp)

                wait_gather(gb, b)
                fire_wb(gb, b)
            return carry

        lax.fori_loop(0, outer, step, 0)
        wait_wb(iters - 1, (iters - 1) % _NB)

    return gk(aux, vtab, idx_flat)


def _attn_body(q8, gaux, gval, auxq,
               p1W, p1b, pg, pb_, p2W, p2b, w1, w1b, wg, wb, w2, w2b,
               sel, ex, out):
    bq = q8.shape[0]
    ga = gaux[...]
    a = _mm(ga, p1W[...]) + p1b[...]
    bqr = _mm(auxq[...], p1W[...])
    t3 = a.reshape(bq, _NS, _C) - bqr[:, None, :]
    t3 = _relu(_ln(t3, pg[...], pb_[...]))
    t = t3.reshape(bq * _NS, _C)
    peb = _mm(t, p2W[...]) + p2b[...]
    u8 = _mm(peb, w1[...]) + _mm(ga, sel[...])
    u3 = u8.reshape(bq, _NS, _G) - q8[...][:, None, :] + w1b[...]
    u3 = _relu(_ln(u3, wg[...], wb[...]))
    w = _mm(u3.reshape(bq * _NS, _G), w2[...]) + w2b[...]
    w3 = w.reshape(bq, _NS, _G)
    w3 = w3 - jnp.max(w3, axis=1, keepdims=True)
    w3 = jnp.exp(w3)
    w3 = w3 / jnp.sum(w3, axis=1, keepdims=True)
    wfull = _mm(w3.reshape(bq * _NS, _G), ex[...])
    acc = (gval[...] + peb) * wfull
    out[...] = jnp.sum(acc.reshape(bq, _NS, _C), axis=1)


def _attn(q8, gaux, gval, auxq, p):
    n = q8.shape[0]
    grid = n // _BQ
    sel = np.zeros((32, _G), np.float32)
    sel[16:16 + _G] = np.eye(_G, dtype=np.float32)
    ex = np.kron(np.eye(_G, dtype=np.float32), np.ones((1, _C // _G), np.float32))
    params = [jnp.concatenate([p["pb1"]["W"].T, jnp.zeros((29, _C), jnp.float32)], axis=0),
              p["pb1"]["b"][None], p["pb_ln"]["g"][None], p["pb_ln"]["b"][None],
              p["pb2"]["W"].T, p["pb2"]["b"][None],
              p["we1"]["W"].T, p["we1"]["b"][None],
              p["we_ln"]["g"][None], p["we_ln"]["b"][None],
              p["we2"]["W"].T, p["we2"]["b"][None], sel, ex]
    full = lambda a: pl.BlockSpec(a.shape, lambda i: (0,) * a.ndim)
    return pl.pallas_call(
        _attn_body,
        grid=(grid,),
        in_specs=[pl.BlockSpec((_BQ, _G), lambda i: (i, 0)),
                  pl.BlockSpec((_BQ * _NS, 32), lambda i: (i, 0)),
                  pl.BlockSpec((_BQ * _NS, _C), lambda i: (i, 0)),
                  pl.BlockSpec((_BQ, 32), lambda i: (i, 0))] + [full(q) for q in params],
        out_specs=pl.BlockSpec((_BQ, _C), lambda i: (i, 0)),
        out_shape=jax.ShapeDtypeStruct((n, _C), jnp.float32),
    )(q8, gaux, gval, auxq, *params)


def _post_sa_body(feat, fsa, n1g, n1b, qW, qb, qg, qlb, w1, feat1_o, q8_o):
    f1 = feat[...] + _ln(fsa[...], n1g[...], n1b[...])
    feat1_o[...] = f1
    q = _relu(_ln(_mm(f1, qW[...]) + qb[...], qg[...], qlb[...]))
    q8_o[...] = _mm(q, w1[...])


def _post_sa(feat, fsa, norm1, ca):
    n = feat.shape[0]
    grid = n // _BP
    row = lambda d: pl.BlockSpec((_BP, d), lambda i: (i, 0))
    full = lambda a: pl.BlockSpec(a.shape, lambda i: (0,) * a.ndim)
    params = [norm1["g"][None], norm1["b"][None],
              ca["q"]["W"].T, ca["q"]["b"][None],
              ca["q_ln"]["g"][None], ca["q_ln"]["b"][None], ca["we1"]["W"].T]
    return pl.pallas_call(
        _post_sa_body,
        grid=(grid,),
        in_specs=[row(_C), row(_C)] + [full(p) for p in params],
        out_specs=[row(_C), row(_G)],
        out_shape=[jax.ShapeDtypeStruct((n, _C), jnp.float32),
                   jax.ShapeDtypeStruct((n, _G), jnp.float32)],
    )(feat, fsa, *params)


def _final_body(feat1, fca, n2g, n2b, m1W, m1b, m2W, m2b, n3g, n3b, out):
    f2 = feat1[...] + _ln(fca[...], n2g[...], n2b[...])
    h = _relu(_mm(f2, m1W[...]) + m1b[...])
    f3 = _ln(_mm(h, m2W[...]) + m2b[...], n3g[...], n3b[...])
    out[...] = _relu(f2 + f3)


def _final(feat1, fca, params):
    n = feat1.shape[0]
    grid = n // _BP
    row = lambda d: pl.BlockSpec((_BP, d), lambda i: (i, 0))
    full = lambda a: pl.BlockSpec(a.shape, lambda i: (0,) * a.ndim)
    ps = [params["norm2"]["g"][None], params["norm2"]["b"][None],
          params["mlp1"]["W"].T, params["mlp1"]["b"][None],
          params["mlp2"]["W"].T, params["mlp2"]["b"][None],
          params["norm3"]["g"][None], params["norm3"]["b"][None]]
    return pl.pallas_call(
        _final_body,
        grid=(grid,),
        in_specs=[row(_C), row(_C)] + [full(p) for p in ps],
        out_specs=row(_C),
        out_shape=jax.ShapeDtypeStruct((n, _C), jnp.float32),
    )(feat1, fca, *ps)


def kernel(coord, feat, offset, reference_index_self_attn, skip_coord,
           skip_feat, skip_offset, reference_index_cross_attn, params):
    n = feat.shape[0]
    z13 = jnp.zeros((n, 13), jnp.float32)
    xyz16 = jnp.concatenate([coord, z13], axis=1)
    sxyz16 = jnp.concatenate([skip_coord, z13], axis=1)

    q8s, auxs, vs, auxc, vc = _proj(feat, skip_feat, xyz16, sxyz16,
                                    params["sa"], params["ca"])

    gaux_s, gval_s = _sc_gather(auxs, vs, reference_index_self_attn.reshape(-1))
    f_sa = _attn(q8s, gaux_s, gval_s, auxs, params["sa"])

    feat1, q8c = _post_sa(feat, f_sa, params["norm1"], params["ca"])

    gaux_c, gval_c = _sc_gather(auxc, vc, reference_index_cross_attn.reshape(-1))
    f_ca = _attn(q8c, gaux_c, gval_c, auxs, params["ca"])

    out_feat = _final(feat1, f_ca, params)
    return (coord, out_feat, offset)


# speedup vs baseline: 5.0930x; 1.0164x over previous
import functools

import numpy as np
import jax
import jax.numpy as jnp
from jax import lax
from jax.experimental import pallas as pl
from jax.experimental.pallas import tpu as pltpu
from jax.experimental.pallas import tpu_sc as plsc

_C = 128
_G = 8
_NS = 32
_BQ = 200
_BP = 2000
_CH = 80


def _mm(x, w):
    return lax.dot_general(x, w, (((x.ndim - 1,), (0,)), ((), ())),
                           preferred_element_type=jnp.float32)


def _ln(x, g, b, eps=1e-5):
    m = jnp.mean(x, axis=-1, keepdims=True)
    v = jnp.mean((x - m) ** 2, axis=-1, keepdims=True)
    return (x - m) / jnp.sqrt(v + eps) * g + b


def _relu(x):
    return jnp.maximum(x, 0.0)


def _proj_body(feat, skip, xyz16, sxyz16,
               qW, qb, qg, qlb, kW, kb, kg, klb, vW, vb, w1,
               ckW, ckb, ckg, cklb, cvW, cvb, cw1,
               q8_o, auxs_o, vs_o, auxc_o, vc_o):
    f = feat[...]
    s = skip[...]
    q = _relu(_ln(_mm(f, qW[...]) + qb[...], qg[...], qlb[...]))
    k = _relu(_ln(_mm(f, kW[...]) + kb[...], kg[...], klb[...]))
    q8_o[...] = _mm(q, w1[...])
    z8 = jnp.zeros((f.shape[0], _G), jnp.float32)
    auxs_o[...] = jnp.concatenate([xyz16[...], _mm(k, w1[...]), z8], axis=1)
    vs_o[...] = _mm(f, vW[...]) + vb[...]
    ck = _relu(_ln(_mm(s, ckW[...]) + ckb[...], ckg[...], cklb[...]))
    auxc_o[...] = jnp.concatenate([sxyz16[...], _mm(ck, cw1[...]), z8], axis=1)
    vc_o[...] = _mm(s, cvW[...]) + cvb[...]


def _proj(feat, skip, xyz16, sxyz16, sa, ca):
    n = feat.shape[0]
    grid = n // _BP
    row = lambda d: pl.BlockSpec((_BP, d), lambda i: (i, 0))
    full = lambda a: pl.BlockSpec(a.shape, lambda i: (0,) * a.ndim)
    params = [sa["q"]["W"].T, sa["q"]["b"][None], sa["q_ln"]["g"][None], sa["q_ln"]["b"][None],
              sa["k"]["W"].T, sa["k"]["b"][None], sa["k_ln"]["g"][None], sa["k_ln"]["b"][None],
              sa["v"]["W"].T, sa["v"]["b"][None], sa["we1"]["W"].T,
              ca["k"]["W"].T, ca["k"]["b"][None], ca["k_ln"]["g"][None], ca["k_ln"]["b"][None],
              ca["v"]["W"].T, ca["v"]["b"][None], ca["we1"]["W"].T]
    return pl.pallas_call(
        _proj_body,
        grid=(grid,),
        in_specs=[row(_C), row(_C), row(16), row(16)] + [full(p) for p in params],
        out_specs=[row(_G), row(32), row(_C), row(32), row(_C)],
        out_shape=[jax.ShapeDtypeStruct((n, _G), jnp.float32),
                   jax.ShapeDtypeStruct((n, 32), jnp.float32),
                   jax.ShapeDtypeStruct((n, _C), jnp.float32),
                   jax.ShapeDtypeStruct((n, 32), jnp.float32),
                   jax.ShapeDtypeStruct((n, _C), jnp.float32)],
    )(feat, skip, xyz16, sxyz16, *params)


_NB = 5


def _sc_gather(aux, vtab, idx_flat):
    r = idx_flat.shape[0]
    info = plsc.get_sparse_core_info()
    nw = info.num_cores * info.num_subcores
    rw = r // nw
    iters = rw // _CH
    outer = iters // _NB
    mesh = plsc.VectorSubcoreMesh(core_axis_name="c", subcore_axis_name="s")

    @functools.partial(
        pl.kernel, mesh=mesh,
        out_type=[jax.ShapeDtypeStruct((r, 32), jnp.float32),
                  jax.ShapeDtypeStruct((r, _C), jnp.float32)],
        scratch_types=[pltpu.VMEM((rw,), jnp.int32),
                       pltpu.VMEM((_NB, _CH, 32), jnp.float32),
                       pltpu.VMEM((_NB, _CH, _C), jnp.float32),
                       pltpu.SemaphoreType.DMA((_NB,)),
                       pltpu.SemaphoreType.DMA((_NB,))],
        compiler_params=pltpu.CompilerParams(use_tc_tiling_on_sc=False),
    )
    def gk(aux_h, vtab_h, idx_h, gaux_h, gval_h, idx_v, abuf, vbuf, semg, semw):
        wid = lax.axis_index("s") * info.num_cores + lax.axis_index("c")
        base = wid * rw
        pltpu.sync_copy(idx_h.at[pl.ds(base, rw)], idx_v)

        def fire(chunk, b):
            sl = idx_v.at[pl.ds(chunk * _CH, _CH)]
            pltpu.async_copy(aux_h.at[sl], abuf.at[b], semg.at[b])
            pltpu.async_copy(vtab_h.at[sl], vbuf.at[b], semg.at[b])

        def wait_gather(chunk, b):
            sl = idx_v.at[pl.ds(chunk * _CH, _CH)]
            pltpu.make_async_copy(aux_h.at[sl], abuf.at[b], semg.at[b]).wait()
            pltpu.make_async_copy(vtab_h.at[sl], vbuf.at[b], semg.at[b]).wait()

        def fire_wb(chunk, b):
            off = base + chunk * _CH
            pltpu.async_copy(abuf.at[b], gaux_h.at[pl.ds(off, _CH)], semw.at[b])
            pltpu.async_copy(vbuf.at[b], gval_h.at[pl.ds(off, _CH)], semw.at[b])

        def wait_wb(chunk, b):
            off = base + chunk * _CH
            pltpu.make_async_copy(abuf.at[b], gaux_h.at[pl.ds(off, _CH)],
                                  semw.at[b]).wait()
            pltpu.make_async_copy(vbuf.at[b], gval_h.at[pl.ds(off, _CH)],
                                  semw.at[b]).wait()

        for b in range(_NB - 1):
            fire(b, b)

        def step(g0, carry):
            for b in range(_NB):
                gb = g0 * _NB + b
                bp = (b - 1) % _NB

                @pl.when(gb > 0)
                def _():
                    wait_wb(gb - 1, bp)

                @pl.when(gb + _NB - 1 < iters)
                def _():
                    fire(gb + _NB - 1, bp)

                wait_gather(gb, b)
                fire_wb(gb, b)
            return carry

        lax.fori_loop(0, outer, step, 0)
        wait_wb(iters - 1, (iters - 1) % _NB)

    return gk(aux, vtab, idx_flat)


def _lnj(x, g, b, j, eps=1e-5):
    m = _mm(x, j)
    xc = x - m
    v = _mm(xc * xc, j)
    return xc * (lax.rsqrt(v + eps) * g) + b


def _attn_core(q8, gaux, gval, auxq,
               p1W, p1b, pg, pb_, p2W, p2b, w1, w1b, wg, wb, w2, w2b,
               sel, ex, j128, j8):
    bq = q8.shape[0]
    ga = gaux[...]
    a = _mm(ga, p1W[...]) + p1b[...]
    bqr = _mm(auxq[...], p1W[...])
    t3 = a.reshape(bq, _NS, _C) - bqr[:, None, :]
    t = _relu(_lnj(t3.reshape(bq * _NS, _C), pg[...], pb_[...], j128[...]))
    peb = _mm(t, p2W[...]) + p2b[...]
    u8 = _mm(peb, w1[...]) + _mm(ga, sel[...])
    u3 = u8.reshape(bq, _NS, _G) - (q8[...] - w1b[...])[:, None, :]
    u = _relu(_lnj(u3.reshape(bq * _NS, _G), wg[...], wb[...], j8[...]))
    w = _mm(u, w2[...]) + w2b[...]
    w3 = w.reshape(bq, _NS, _G)
    w3 = w3 - jnp.max(w3, axis=1, keepdims=True)
    w3 = jnp.exp(w3)
    w3 = w3 / jnp.sum(w3, axis=1, keepdims=True)
    wfull = _mm(w3.reshape(bq * _NS, _G), ex[...])
    acc = (gval[...] + peb) * wfull
    return jnp.sum(acc.reshape(bq, _NS, _C), axis=1)


def _attn_params(p):
    sel = np.zeros((32, _G), np.float32)
    sel[16:16 + _G] = np.eye(_G, dtype=np.float32)
    ex = np.kron(np.eye(_G, dtype=np.float32), np.ones((1, _C // _G), np.float32))
    return [jnp.concatenate([p["pb1"]["W"].T, jnp.zeros((29, _C), jnp.float32)], axis=0),
            p["pb1"]["b"][None], p["pb_ln"]["g"][None], p["pb_ln"]["b"][None],
            p["pb2"]["W"].T, p["pb2"]["b"][None],
            p["we1"]["W"].T, p["we1"]["b"][None],
            p["we_ln"]["g"][None], p["we_ln"]["b"][None],
            p["we2"]["W"].T, p["we2"]["b"][None], sel, ex,
            np.full((_C, _C), 1.0 / _C, np.float32),
            np.full((_G, _G), 1.0 / _G, np.float32)]


def _attn_sa_body(q8, gaux, gval, auxq, feat,
                  p1W, p1b, pg, pb_, p2W, p2b, w1, w1b, wg, wb, w2, w2b,
                  sel, ex, j128, j8,
                  n1g, n1b, qW, qb, qg, qlb, cw1,
                  feat1_o, q8c_o):
    f = _attn_core(q8, gaux, gval, auxq, p1W, p1b, pg, pb_, p2W, p2b,
                   w1, w1b, wg, wb, w2, w2b, sel, ex, j128, j8)
    f1 = feat[...] + _lnj(f, n1g[...], n1b[...], j128[...])
    feat1_o[...] = f1
    q = _relu(_lnj(_mm(f1, qW[...]) + qb[...], qg[...], qlb[...], j128[...]))
    q8c_o[...] = _mm(q, cw1[...])


def _attn_sa(q8, gaux, gval, auxq, feat, sa, norm1, ca):
    n = q8.shape[0]
    grid = n // _BQ
    params = _attn_params(sa) + [
        norm1["g"][None], norm1["b"][None],
        ca["q"]["W"].T, ca["q"]["b"][None],
        ca["q_ln"]["g"][None], ca["q_ln"]["b"][None], ca["we1"]["W"].T]
    full = lambda a: pl.BlockSpec(a.shape, lambda i: (0,) * a.ndim)
    return pl.pallas_call(
        _attn_sa_body,
        grid=(grid,),
        in_specs=[pl.BlockSpec((_BQ, _G), lambda i: (i, 0)),
                  pl.BlockSpec((_BQ * _NS, 32), lambda i: (i, 0)),
                  pl.BlockSpec((_BQ * _NS, _C), lambda i: (i, 0)),
                  pl.BlockSpec((_BQ, 32), lambda i: (i, 0)),
                  pl.BlockSpec((_BQ, _C), lambda i: (i, 0))] + [full(q) for q in params],
        out_specs=[pl.BlockSpec((_BQ, _C), lambda i: (i, 0)),
                   pl.BlockSpec((_BQ, _G), lambda i: (i, 0))],
        out_shape=[jax.ShapeDtypeStruct((n, _C), jnp.float32),
                   jax.ShapeDtypeStruct((n, _G), jnp.float32)],
    )(q8, gaux, gval, auxq, feat, *params)


def _attn_ca_body(q8, gaux, gval, auxq, feat1,
                  p1W, p1b, pg, pb_, p2W, p2b, w1, w1b, wg, wb, w2, w2b,
                  sel, ex, j128, j8,
                  n2g, n2b, m1W, m1b, m2W, m2b, n3g, n3b,
                  out):
    f = _attn_core(q8, gaux, gval, auxq, p1W, p1b, pg, pb_, p2W, p2b,
                   w1, w1b, wg, wb, w2, w2b, sel, ex, j128, j8)
    f2 = feat1[...] + _lnj(f, n2g[...], n2b[...], j128[...])
    h = _relu(_mm(f2, m1W[...]) + m1b[...])
    f3 = _lnj(_mm(h, m2W[...]) + m2b[...], n3g[...], n3b[...], j128[...])
    out[...] = _relu(f2 + f3)


def _attn_ca(q8, gaux, gval, auxq, feat1, ca, params):
    n = q8.shape[0]
    grid = n // _BQ
    ps = _attn_params(ca) + [
        params["norm2"]["g"][None], params["norm2"]["b"][None],
        params["mlp1"]["W"].T, params["mlp1"]["b"][None],
        params["mlp2"]["W"].T, params["mlp2"]["b"][None],
        params["norm3"]["g"][None], params["norm3"]["b"][None]]
    full = lambda a: pl.BlockSpec(a.shape, lambda i: (0,) * a.ndim)
    return pl.pallas_call(
        _attn_ca_body,
        grid=(grid,),
        in_specs=[pl.BlockSpec((_BQ, _G), lambda i: (i, 0)),
                  pl.BlockSpec((_BQ * _NS, 32), lambda i: (i, 0)),
                  pl.BlockSpec((_BQ * _NS, _C), lambda i: (i, 0)),
                  pl.BlockSpec((_BQ, 32), lambda i: (i, 0)),
                  pl.BlockSpec((_BQ, _C), lambda i: (i, 0))] + [full(q) for q in ps],
        out_specs=pl.BlockSpec((_BQ, _C), lambda i: (i, 0)),
        out_shape=jax.ShapeDtypeStruct((n, _C), jnp.float32),
    )(q8, gaux, gval, auxq, feat1, *ps)


def kernel(coord, feat, offset, reference_index_self_attn, skip_coord,
           skip_feat, skip_offset, reference_index_cross_attn, params):
    n = feat.shape[0]
    z13 = jnp.zeros((n, 13), jnp.float32)
    xyz16 = jnp.concatenate([coord, z13], axis=1)
    sxyz16 = jnp.concatenate([skip_coord, z13], axis=1)

    q8s, auxs, vs, auxc, vc = _proj(feat, skip_feat, xyz16, sxyz16,
                                    params["sa"], params["ca"])

    gaux_s, gval_s = _sc_gather(auxs, vs, reference_index_self_attn.reshape(-1))
    gaux_c, gval_c = _sc_gather(auxc, vc, reference_index_cross_attn.reshape(-1))
    feat1, q8c = _attn_sa(q8s, gaux_s, gval_s, auxs, feat,
                          params["sa"], params["norm1"], params["ca"])
    out_feat = _attn_ca(q8c, gaux_c, gval_c, auxs, feat1, params["ca"], params)
    return (coord, out_feat, offset)


# speedup vs baseline: 5.2879x; 1.0383x over previous
import functools

import numpy as np
import jax
import jax.numpy as jnp
from jax import lax
from jax.experimental import pallas as pl
from jax.experimental.pallas import tpu as pltpu
from jax.experimental.pallas import tpu_sc as plsc

_C = 128
_G = 8
_NS = 32
_BQ = 200
_BP = 2000
_CH = 40
_NB = 5

_EX = np.kron(np.eye(_G, dtype=np.float32), np.ones((1, _C // _G), np.float32))


def _mm(x, w):
    return lax.dot_general(x, w, (((x.ndim - 1,), (0,)), ((), ())),
                           preferred_element_type=jnp.float32)


def _relu(x):
    return jnp.maximum(x, 0.0)


def _lnj(x, g, b, j, eps=1e-5):
    m = _mm(x, j)
    xc = x - m
    v = _mm(xc * xc, j)
    return xc * (lax.rsqrt(v + eps) * g) + b


def _proj_body(feat, skip, xyz16, sxyz16, j128,
               qW, qb, qg, qlb, kW, kb, kg, klb, vW, vb, w1r, p1s,
               ckW, ckb, ckg, cklb, cvW, cvb, cw1r, cp1,
               q8_o, pts_o, k8s_o, vs_o, ptqc_o, ptkc_o, k8c_o, vc_o):
    f = feat[...]
    s = skip[...]
    j = j128[...]
    q = _relu(_lnj(_mm(f, qW[...]) + qb[...], qg[...], qlb[...], j))
    k = _relu(_lnj(_mm(f, kW[...]) + kb[...], kg[...], klb[...], j))
    q8_o[...] = _mm(q, w1r[...])
    k8s_o[...] = _mm(k, w1r[...])
    vs_o[...] = _mm(f, vW[...]) + vb[...]
    pts_o[...] = _mm(xyz16[...], p1s[...])
    ptqc_o[...] = _mm(xyz16[...], cp1[...])
    ptkc_o[...] = _mm(sxyz16[...], cp1[...])
    ck = _relu(_lnj(_mm(s, ckW[...]) + ckb[...], ckg[...], cklb[...], j))
    k8c_o[...] = _mm(ck, cw1r[...])
    vc_o[...] = _mm(s, cvW[...]) + cvb[...]


def _proj(feat, skip, xyz16, sxyz16, sa, ca, w1r_s, w1r_c):
    n = feat.shape[0]
    grid = n // _BP
    row = lambda d: pl.BlockSpec((_BP, d), lambda i: (i, 0))
    full = lambda a: pl.BlockSpec(a.shape, lambda i: (0,) * a.ndim)
    p1s = jnp.concatenate([sa["pb1"]["W"].T, jnp.zeros((13, _C), jnp.float32)], axis=0)
    cp1 = jnp.concatenate([ca["pb1"]["W"].T, jnp.zeros((13, _C), jnp.float32)], axis=0)
    params = [np.full((_C, _C), 1.0 / _C, np.float32),
              sa["q"]["W"].T, sa["q"]["b"][None], sa["q_ln"]["g"][None], sa["q_ln"]["b"][None],
              sa["k"]["W"].T, sa["k"]["b"][None], sa["k_ln"]["g"][None], sa["k_ln"]["b"][None],
              sa["v"]["W"].T, sa["v"]["b"][None], w1r_s, p1s,
              ca["k"]["W"].T, ca["k"]["b"][None], ca["k_ln"]["g"][None], ca["k_ln"]["b"][None],
              ca["v"]["W"].T, ca["v"]["b"][None], w1r_c, cp1]
    big = jax.ShapeDtypeStruct((n, _C), jnp.float32)
    return pl.pallas_call(
        _proj_body,
        grid=(grid,),
        in_specs=[row(_C), row(_C), row(16), row(16)] + [full(p) for p in params],
        out_specs=[row(_C)] * 8,
        out_shape=[big] * 8,
    )(feat, skip, xyz16, sxyz16, *params)


def _sc_gather(pt, vtab, ktab, idx_flat):
    r = idx_flat.shape[0]
    info = plsc.get_sparse_core_info()
    nw = info.num_cores * info.num_subcores
    rw = r // nw
    iters = rw // _CH
    outer = iters // _NB
    mesh = plsc.VectorSubcoreMesh(core_axis_name="c", subcore_axis_name="s")
    big = jax.ShapeDtypeStruct((r, _C), jnp.float32)

    @functools.partial(
        pl.kernel, mesh=mesh,
        out_type=[big, big, big],
        scratch_types=[pltpu.VMEM((rw,), jnp.int32),
                       pltpu.VMEM((_NB, _CH, _C), jnp.float32),
                       pltpu.VMEM((_NB, _CH, _C), jnp.float32),
                       pltpu.VMEM((_NB, _CH, _C), jnp.float32),
                       pltpu.SemaphoreType.DMA((_NB,)),
                       pltpu.SemaphoreType.DMA((_NB,))],
        compiler_params=pltpu.CompilerParams(use_tc_tiling_on_sc=False),
    )
    def gk(pt_h, vtab_h, ktab_h, idx_h, gp_h, gv_h, gk_h,
           idx_v, pbuf, vbuf, kbuf, semg, semw):
        wid = lax.axis_index("s") * info.num_cores + lax.axis_index("c")
        base = wid * rw
        pltpu.sync_copy(idx_h.at[pl.ds(base, rw)], idx_v)

        def fire(chunk, b):
            sl = idx_v.at[pl.ds(chunk * _CH, _CH)]
            pltpu.async_copy(pt_h.at[sl], pbuf.at[b], semg.at[b])
            pltpu.async_copy(vtab_h.at[sl], vbuf.at[b], semg.at[b])
            pltpu.async_copy(ktab_h.at[sl], kbuf.at[b], semg.at[b])

        def wait_gather(chunk, b):
            sl = idx_v.at[pl.ds(chunk * _CH, _CH)]
            pltpu.make_async_copy(pt_h.at[sl], pbuf.at[b], semg.at[b]).wait()
            pltpu.make_async_copy(vtab_h.at[sl], vbuf.at[b], semg.at[b]).wait()
            pltpu.make_async_copy(ktab_h.at[sl], kbuf.at[b], semg.at[b]).wait()

        def fire_wb(chunk, b):
            off = base + chunk * _CH
            pltpu.async_copy(pbuf.at[b], gp_h.at[pl.ds(off, _CH)], semw.at[b])
            pltpu.async_copy(vbuf.at[b], gv_h.at[pl.ds(off, _CH)], semw.at[b])
            pltpu.async_copy(kbuf.at[b], gk_h.at[pl.ds(off, _CH)], semw.at[b])

        def wait_wb(chunk, b):
            off = base + chunk * _CH
            pltpu.make_async_copy(pbuf.at[b], gp_h.at[pl.ds(off, _CH)],
                                  semw.at[b]).wait()
            pltpu.make_async_copy(vbuf.at[b], gv_h.at[pl.ds(off, _CH)],
                                  semw.at[b]).wait()
            pltpu.make_async_copy(kbuf.at[b], gk_h.at[pl.ds(off, _CH)],
                                  semw.at[b]).wait()

        for b in range(_NB - 1):
            fire(b, b)

        def step(g0, carry):
            for b in range(_NB):
                gb = g0 * _NB + b
                bp = (b - 1) % _NB

                @pl.when(gb > 0)
                def _():
                    wait_wb(gb - 1, bp)

                @pl.when(gb + _NB - 1 < iters)
                def _():
                    fire(gb + _NB - 1, bp)

                wait_gather(gb, b)
                fire_wb(gb, b)
            return carry

        lax.fori_loop(0, outer, step, 0)
        wait_wb(iters - 1, (iters - 1) % _NB)

    return gk(pt, vtab, ktab, idx_flat)


def _attn_core(q8, gp, gval, gk8, ptq,
               p1b, pg, pb_, p2W, p2b, w1r, w1br, wgr, wbr, m2r, w2br,
               j128):
    bq = q8.shape[0]
    j = j128[...]
    a = gp[...] + p1b[...]
    t3 = a.reshape(bq, _NS, _C) - ptq[...][:, None, :]
    t = _relu(_lnj(t3.reshape(bq * _NS, _C), pg[...], pb_[...], j))
    peb = _mm(t, p2W[...]) + p2b[...]
    u = _mm(peb, w1r[...]) + gk8[...]
    q8r = q8[...] - w1br[...]
    u3 = u.reshape(bq, _NS, _C) - q8r[:, None, :]
    u = _relu(_lnj(u3.reshape(bq * _NS, _C), wgr[...], wbr[...], j))
    w = _mm(u, m2r[...]) + w2br[...]
    w3 = w.reshape(bq, _NS, _C)
    w3 = w3 - jnp.max(w3, axis=1, keepdims=True)
    w3 = jnp.exp(w3)
    w3 = w3 / jnp.sum(w3, axis=1, keepdims=True)
    acc = (gval[...] + peb) * w3.reshape(bq * _NS, _C)
    return jnp.sum(acc.reshape(bq, _NS, _C), axis=1)


def _attn_params(p, w1r):
    ex = jnp.asarray(_EX)
    return [p["pb1"]["b"][None], p["pb_ln"]["g"][None], p["pb_ln"]["b"][None],
            p["pb2"]["W"].T, p["pb2"]["b"][None],
            w1r, p["we1"]["b"][None] @ ex,
            p["we_ln"]["g"][None] @ ex, p["we_ln"]["b"][None] @ ex,
            jnp.kron(p["we2"]["W"].T, jnp.full((16, 16), 1.0 / 16, jnp.float32)),
            p["we2"]["b"][None] @ ex,
            np.full((_C, _C), 1.0 / _C, np.float32)]


def _attn_sa(q8, gp, gval, gk8, ptq, feat, sa, w1r_s, norm1, ca, w1r_c):
    n = q8.shape[0]
    grid = n // _BQ

    def body(q8_r, gp_r, gval_r, gk8_r, ptq_r, feat_r,
             p1b, pg, pb_, p2W, p2b, w1r, w1br, wgr, wbr, m2r, w2br, j128,
             n1g, n1b, qW, qb, qg, qlb, cw1r,
             feat1_o, q8c_o):
        f = _attn_core(q8_r, gp_r, gval_r, gk8_r, ptq_r,
                       p1b, pg, pb_, p2W, p2b, w1r, w1br, wgr, wbr, m2r, w2br,
                       j128)
        j = j128[...]
        f1 = feat_r[...] + _lnj(f, n1g[...], n1b[...], j)
        feat1_o[...] = f1
        q = _relu(_lnj(_mm(f1, qW[...]) + qb[...], qg[...], qlb[...], j))
        q8c_o[...] = _mm(q, cw1r[...])

    params = _attn_params(sa, w1r_s) + [
        norm1["g"][None], norm1["b"][None],
        ca["q"]["W"].T, ca["q"]["b"][None],
        ca["q_ln"]["g"][None], ca["q_ln"]["b"][None], w1r_c]
    full = lambda a: pl.BlockSpec(a.shape, lambda i: (0,) * a.ndim)
    gb = lambda: pl.BlockSpec((_BQ * _NS, _C), lambda i: (i, 0))
    return pl.pallas_call(
        body,
        grid=(grid,),
        in_specs=[pl.BlockSpec((_BQ, _C), lambda i: (i, 0)),
                  gb(), gb(), gb(),
                  pl.BlockSpec((_BQ, _C), lambda i: (i, 0)),
                  pl.BlockSpec((_BQ, _C), lambda i: (i, 0))] + [full(q) for q in params],
        out_specs=[pl.BlockSpec((_BQ, _C), lambda i: (i, 0)),
                   pl.BlockSpec((_BQ, _C), lambda i: (i, 0))],
        out_shape=[jax.ShapeDtypeStruct((n, _C), jnp.float32),
                   jax.ShapeDtypeStruct((n, _C), jnp.float32)],
    )(q8, gp, gval, gk8, ptq, feat, *params)


def _attn_ca(q8, gp, gval, gk8, ptq, feat1, ca, w1r_c, params):
    n = q8.shape[0]
    grid = n // _BQ

    def body(q8_r, gp_r, gval_r, gk8_r, ptq_r, feat1_r,
             p1b, pg, pb_, p2W, p2b, w1r, w1br, wgr, wbr, m2r, w2br, j128,
             n2g, n2b, m1W, m1b, m2W, m2b, n3g, n3b,
             out):
        f = _attn_core(q8_r, gp_r, gval_r, gk8_r, ptq_r,
                       p1b, pg, pb_, p2W, p2b, w1r, w1br, wgr, wbr, m2r, w2br,
                       j128)
        j = j128[...]
        f2 = feat1_r[...] + _lnj(f, n2g[...], n2b[...], j)
        h = _relu(_mm(f2, m1W[...]) + m1b[...])
        f3 = _lnj(_mm(h, m2W[...]) + m2b[...], n3g[...], n3b[...], j)
        out[...] = _relu(f2 + f3)

    ps = _attn_params(ca, w1r_c) + [
        params["norm2"]["g"][None], params["norm2"]["b"][None],
        params["mlp1"]["W"].T, params["mlp1"]["b"][None],
        params["mlp2"]["W"].T, params["mlp2"]["b"][None],
        params["norm3"]["g"][None], params["norm3"]["b"][None]]
    full = lambda a: pl.BlockSpec(a.shape, lambda i: (0,) * a.ndim)
    gb = lambda: pl.BlockSpec((_BQ * _NS, _C), lambda i: (i, 0))
    return pl.pallas_call(
        body,
        grid=(grid,),
        in_specs=[pl.BlockSpec((_BQ, _C), lambda i: (i, 0)),
                  gb(), gb(), gb(),
                  pl.BlockSpec((_BQ, _C), lambda i: (i, 0)),
                  pl.BlockSpec((_BQ, _C), lambda i: (i, 0))] + [full(q) for q in ps],
        out_specs=pl.BlockSpec((_BQ, _C), lambda i: (i, 0)),
        out_shape=jax.ShapeDtypeStruct((n, _C), jnp.float32),
    )(q8, gp, gval, gk8, ptq, feat1, *ps)


def kernel(coord, feat, offset, reference_index_self_attn, skip_coord,
           skip_feat, skip_offset, reference_index_cross_attn, params):
    n = feat.shape[0]
    sa, ca = params["sa"], params["ca"]
    z13 = jnp.zeros((n, 13), jnp.float32)
    xyz16 = jnp.concatenate([coord, z13], axis=1)
    sxyz16 = jnp.concatenate([skip_coord, z13], axis=1)
    ex = jnp.asarray(_EX)
    w1r_s = sa["we1"]["W"].T @ ex
    w1r_c = ca["we1"]["W"].T @ ex

    (q8s, pts, k8s, vs, ptqc, ptkc, k8c, vc) = _proj(
        feat, skip_feat, xyz16, sxyz16, sa, ca, w1r_s, w1r_c)

    gp_s, gv_s, gk_s = _sc_gather(pts, vs, k8s,
                                  reference_index_self_attn.reshape(-1))
    gp_c, gv_c, gk_c = _sc_gather(ptkc, vc, k8c,
                                  reference_index_cross_attn.reshape(-1))
    feat1, q8c = _attn_sa(q8s, gp_s, gv_s, gk_s, pts, feat,
                          sa, w1r_s, params["norm1"], ca, w1r_c)
    out_feat = _attn_ca(q8c, gp_c, gv_c, gk_c, ptqc, feat1, ca, w1r_c, params)
    return (coord, out_feat, offset)
